# baseline (speedup 1.0000x reference)
import jax
import jax.numpy as jnp
from jax.experimental import pallas as pl

_C = 192
_H = 32
_W = 32
_N = _H * _W
_NPAD = 1152
_OH = 30
_NEG = -3.0e38


def _match_kernel(f1_ref, f2_ref, flow_ref, off_ref, sim_ref):
    f1 = f1_ref[0]
    f2 = f2_ref[0]

    n1 = jnp.sqrt(jnp.sum(f1 * f1, axis=0, keepdims=True))
    fi = f1 / jnp.maximum(n1, 1e-12)
    n2 = jnp.sqrt(jnp.sum(f2 * f2, axis=0, keepdims=True))
    fr = f2 / jnp.maximum(n2, 1e-12)

    sqi = jnp.sum(fi * fi, axis=0, keepdims=True)
    sqr = jnp.sum(fr * fr, axis=0, keepdims=True)

    rn2 = jnp.zeros((1, _N), jnp.float32)
    in2 = jnp.zeros((1, _N), jnp.float32)
    for di in range(3):
        for dj in range(3):
            o = di * _W + dj
            rn2 = rn2 + jax.lax.slice(sqr, (0, o), (1, o + _N))
            in2 = in2 + jax.lax.slice(sqi, (0, o), (1, o + _N))
    rn = jnp.sqrt(rn2) + 1e-5

    acc = jnp.zeros((_N, _N), jnp.float32)
    for di in range(3):
        for dj in range(3):
            o = di * _W + dj
            a = jax.lax.slice(fi, (0, o), (_C, o + _N))
            b = jax.lax.slice(fr, (0, o), (_C, o + _N)) / rn
            acc = acc + jax.lax.dot_general(
                a, b, (((0,), (0,)), ((), ())),
                preferred_element_type=jnp.float32)

    col = jax.lax.broadcasted_iota(jnp.int32, (1, _N), 1)
    colvalid = ((col % _W) < _OH) & ((col // _W) < _OH)
    corr = jnp.where(colvalid, acc, _NEG)

    maxval = jnp.max(corr, axis=1, keepdims=True)
    lane = jax.lax.broadcasted_iota(jnp.int32, (_N, _N), 1)
    idx = jnp.min(jnp.where(corr == maxval, lane, jnp.int32(1 << 30)),
                  axis=1, keepdims=True)

    r0 = jax.lax.broadcasted_iota(jnp.int32, (_N, _N), 0)
    ident = (r0 == lane).astype(jnp.float32)
    in2col = jax.lax.dot_general(
        ident, in2, (((1,), (1,)), ((), ())),
        preferred_element_type=jnp.float32)

    sim = maxval / (jnp.sqrt(in2col) + 1e-5)

    row = jax.lax.broadcasted_iota(jnp.int32, (_N, 1), 0)
    qx = row % _W
    qy = row // _W
    rvalid = (qx < _OH) & (qy < _OH)
    fx = jnp.where(rvalid, (idx % _W - qx).astype(jnp.float32), 0.0)
    fy = jnp.where(rvalid, (idx // _W - qy).astype(jnp.float32), 0.0)
    flow = jnp.concatenate([fx, fy], axis=1)
    flow_ref[0] = flow

    for i in range(3):
        for j in range(3):
            s = i * _W + j
            if s == 0:
                off_ref[0, 0] = flow
            else:
                off_ref[0, i * 3 + j] = jnp.concatenate(
                    [jnp.zeros((s, 2), jnp.float32),
                     jax.lax.slice(flow, (0, 0), (_N - s, 2))], axis=0)

    simz = jnp.where(rvalid, sim, 0.0)
    sim_ref[0] = jnp.concatenate(
        [jnp.zeros((_H + 1, 1), jnp.float32),
         jax.lax.slice(simz, (0, 0), (_N - _H - 1, 1))], axis=0)


@jax.jit
def kernel(features1, features2):
    b = features1.shape[0]
    f1 = jnp.pad(features1.reshape(b, _C, _N), ((0, 0), (0, 0), (0, _NPAD - _N)))
    f2 = jnp.pad(features2.reshape(b, _C, _N), ((0, 0), (0, 0), (0, _NPAD - _N)))

    flow, off, sim = pl.pallas_call(
        _match_kernel,
        grid=(b,),
        in_specs=[
            pl.BlockSpec((1, _C, _NPAD), lambda i: (i, 0, 0)),
            pl.BlockSpec((1, _C, _NPAD), lambda i: (i, 0, 0)),
        ],
        out_specs=[
            pl.BlockSpec((1, _N, 2), lambda i: (i, 0, 0)),
            pl.BlockSpec((1, 9, _N, 2), lambda i: (i, 0, 0, 0)),
            pl.BlockSpec((1, _N, 1), lambda i: (i, 0, 0)),
        ],
        out_shape=[
            jax.ShapeDtypeStruct((b, _N, 2), jnp.float32),
            jax.ShapeDtypeStruct((b, 9, _N, 2), jnp.float32),
            jax.ShapeDtypeStruct((b, _N, 1), jnp.float32),
        ],
    )(f1, f2)

    pre_flow = flow.reshape(b, _H, _W, 2)
    pre_offset = off.reshape(b, 9, _H, _W, 2)
    pre_similarity = sim.reshape(b, 1, _H, _W)
    return (pre_flow, pre_offset, pre_similarity)


# speedup vs baseline: 1.3523x; 1.3523x over previous
import jax
import jax.numpy as jnp
from jax.experimental import pallas as pl

_C = 192
_H = 32
_W = 32
_N = _H * _W
_OH = 30
_K = 9 * _C
_NEG = -3.0e38
_OFFS = tuple(di * _W + dj for di in range(3) for dj in range(3))


def _shl(v, o, rows):
    if o == 0:
        return v
    return jnp.concatenate(
        [jax.lax.slice(v, (0, o), (rows, _N)), jnp.zeros((rows, o), v.dtype)],
        axis=1)


def _shr(v, o, rows):
    if o == 0:
        return v
    return jnp.concatenate(
        [jnp.zeros((rows, o), v.dtype), jax.lax.slice(v, (0, 0), (rows, _N - o))],
        axis=1)


def _match_kernel(f1_ref, f2_ref, off_ref, sim_ref):
    f1 = f1_ref[0]
    f2 = f2_ref[0]

    n1 = jnp.sqrt(jnp.sum(f1 * f1, axis=0, keepdims=True))
    fi = f1 / jnp.maximum(n1, 1e-12)
    n2 = jnp.sqrt(jnp.sum(f2 * f2, axis=0, keepdims=True))
    fr = f2 / jnp.maximum(n2, 1e-12)

    sqi = jnp.sum(fi * fi, axis=0, keepdims=True)
    sqr = jnp.sum(fr * fr, axis=0, keepdims=True)

    rn2 = _shl(sqr, _OFFS[0], 1)
    in2 = _shl(sqi, _OFFS[0], 1)
    for o in _OFFS[1:]:
        rn2 = rn2 + _shl(sqr, o, 1)
        in2 = in2 + _shl(sqi, o, 1)
    rn = jnp.sqrt(rn2) + 1e-5

    a_cat = jnp.concatenate([_shl(fi, o, _C) for o in _OFFS], axis=0)
    b_cat = jnp.concatenate([_shl(fr, o, _C) for o in _OFFS], axis=0) / rn
    acc = jax.lax.dot_general(
        a_cat, b_cat, (((0,), (0,)), ((), ())),
        preferred_element_type=jnp.float32)

    colx = jax.lax.broadcasted_iota(jnp.int32, (1, _N), 1)
    colvalid = ((colx % _W) < _OH) & ((colx // _W) < _OH)
    corr = jnp.where(colvalid, acc, _NEG)

    maxval = jnp.max(corr, axis=1, keepdims=True)
    lane = jax.lax.broadcasted_iota(jnp.int32, (_N, _N), 1)
    idx = jnp.min(jnp.where(corr == maxval, lane, jnp.int32(1 << 30)),
                  axis=1, keepdims=True)

    pack = jnp.concatenate(
        [(idx % _W).astype(jnp.float32),
         (idx // _W).astype(jnp.float32),
         maxval], axis=1)
    r0 = jax.lax.broadcasted_iota(jnp.int32, (_N, _N), 0)
    ident = (r0 == lane).astype(jnp.float32)
    rows = jax.lax.dot_general(
        pack, ident, (((0,), (0,)), ((), ())),
        preferred_element_type=jnp.float32,
        precision=jax.lax.Precision.HIGHEST)

    pxr = jax.lax.slice(rows, (0, 0), (1, _N))
    pyr = jax.lax.slice(rows, (1, 0), (2, _N))
    maxr = jax.lax.slice(rows, (2, 0), (3, _N))

    colxf = (colx % _W).astype(jnp.float32)
    colyf = (colx // _W).astype(jnp.float32)
    fxr = jnp.where(colvalid, pxr - colxf, 0.0)
    fyr = jnp.where(colvalid, pyr - colyf, 0.0)

    simr = jnp.where(colvalid, maxr / (jnp.sqrt(in2) + 1e-5), 0.0)

    off_rows = []
    for s in _OFFS:
        off_rows.append(_shr(fxr, s, 1))
        off_rows.append(_shr(fyr, s, 1))
    off_ref[0] = jnp.concatenate(off_rows, axis=0)

    sim_ref[0] = _shr(simr, _W + 1, 1)


@jax.jit
def kernel(features1, features2):
    b = features1.shape[0]
    f1 = features1.reshape(b, _C, _N)
    f2 = features2.reshape(b, _C, _N)

    off, sim = pl.pallas_call(
        _match_kernel,
        grid=(b,),
        in_specs=[
            pl.BlockSpec((1, _C, _N), lambda i: (i, 0, 0)),
            pl.BlockSpec((1, _C, _N), lambda i: (i, 0, 0)),
        ],
        out_specs=[
            pl.BlockSpec((1, 18, _N), lambda i: (i, 0, 0)),
            pl.BlockSpec((1, 1, _N), lambda i: (i, 0, 0)),
        ],
        out_shape=[
            jax.ShapeDtypeStruct((b, 18, _N), jnp.float32),
            jax.ShapeDtypeStruct((b, 1, _N), jnp.float32),
        ],
    )(f1, f2)

    off5 = off.reshape(b, 9, 2, _N).transpose(0, 1, 3, 2)
    pre_offset = off5.reshape(b, 9, _H, _W, 2)
    pre_flow = off5[:, 0].reshape(b, _H, _W, 2)
    pre_similarity = sim.reshape(b, 1, _H, _W)
    return (pre_flow, pre_offset, pre_similarity)
